# baseline (speedup 1.0000x reference)
import functools

import jax
import jax.numpy as jnp
from jax import lax
from jax.experimental import pallas as pl
from jax.experimental.pallas import tpu as pltpu
from jax.experimental.pallas import tpu_sc as plsc

BATCH = 4096
SEQ = 50
EMB = 100

NC = 2
NS = 16
NW = NC * NS

BATCHES_PER_TILE = BATCH // NW
TOKENS_PER_TILE = BATCHES_PER_TILE * SEQ
IDX_PAD = TOKENS_PER_TILE + 16
NBUF = 8


@functools.cache
def _build_gather_kernel():
    mesh = plsc.VectorSubcoreMesh(core_axis_name="c", subcore_axis_name="s")

    @functools.partial(
        pl.kernel,
        out_type=jax.ShapeDtypeStruct((BATCH, SEQ, EMB), jnp.float32),
        mesh=mesh,
        scratch_types=[
            pltpu.VMEM((IDX_PAD,), jnp.int32),
            pltpu.VMEM((NBUF, SEQ, EMB), jnp.float32),
            [pltpu.SemaphoreType.DMA] * NBUF,
            [pltpu.SemaphoreType.DMA] * NBUF,
            [pltpu.SemaphoreType.DMA] * NBUF,
        ],
    )
    def k(idx_hbm, table_hbm, out_hbm, idx_v, bufs, gsems, gsems2, wsems):
        wid = lax.axis_index("s") * NC + lax.axis_index("c")
        b_first = wid * BATCHES_PER_TILE

        pltpu.sync_copy(idx_hbm.at[wid], idx_v.at[pl.ds(0, TOKENS_PER_TILE)])

        def issue_gathers(b, j):
            base = b * SEQ
            vecs = [idx_v[pl.ds(base + off, 16)] for off in (0, 16, 32, 48)]
            for t in range(SEQ):
                v, lane = divmod(t, 16)
                sem = gsems[j] if t % 2 == 0 else gsems2[j]
                pltpu.async_copy(
                    table_hbm.at[pl.ds(vecs[v][lane], 1)],
                    bufs.at[j, pl.ds(t, 1)], sem)

        for j in range(NBUF):
            issue_gathers(j, j)

        def body(i, carry):
            del carry
            for j in range(NBUF):
                b = NBUF * i + j
                for sem in (gsems[j], gsems2[j]):
                    for n in (8, 8, 8, 1):
                        pltpu.make_async_copy(
                            out_hbm.at[0, pl.ds(0, n)],
                            bufs.at[0, pl.ds(0, n)], sem).wait()
                pltpu.async_copy(bufs.at[j], out_hbm.at[b_first + b], wsems[j])
            for j in range(NBUF):
                @pl.when(i < BATCHES_PER_TILE // NBUF - 1)
                def _():
                    pltpu.make_async_copy(
                        bufs.at[0], out_hbm.at[0], wsems[j]).wait()
                    issue_gathers(NBUF * (i + 1) + j, j)
            return 0

        lax.fori_loop(0, BATCHES_PER_TILE // NBUF, body, 0, unroll=False)
        for j in range(NBUF):
            pltpu.make_async_copy(bufs.at[0], out_hbm.at[0], wsems[j]).wait()

    return k


def kernel(indices, table):
    idx = indices.astype(jnp.int32).reshape(NW, TOKENS_PER_TILE)
    return _build_gather_kernel()(idx, table)


# speedup vs baseline: 1.0041x; 1.0041x over previous
import functools

import jax
import jax.numpy as jnp
from jax import lax
from jax.experimental import pallas as pl
from jax.experimental.pallas import tpu as pltpu
from jax.experimental.pallas import tpu_sc as plsc

BATCH = 4096
SEQ = 50
EMB = 100

NC = 2
NS = 16
NW = NC * NS

BATCHES_PER_TILE = BATCH // NW
TOKENS_PER_TILE = BATCHES_PER_TILE * SEQ
IDX_PAD = TOKENS_PER_TILE + 16
NBUF = 8


@functools.cache
def _build_gather_kernel():
    mesh = plsc.VectorSubcoreMesh(core_axis_name="c", subcore_axis_name="s")

    @functools.partial(
        pl.kernel,
        out_type=jax.ShapeDtypeStruct((BATCH, SEQ, EMB), jnp.float32),
        mesh=mesh,
        scratch_types=[
            pltpu.VMEM((IDX_PAD,), jnp.int32),
            pltpu.VMEM((NBUF, SEQ, EMB), jnp.float32),
            [pltpu.SemaphoreType.DMA] * NBUF,
            [pltpu.SemaphoreType.DMA] * NBUF,
        ],
    )
    def k(idx_hbm, table_hbm, out_hbm, idx_v, bufs, gsems, wsems):
        wid = lax.axis_index("s") * NC + lax.axis_index("c")
        b_first = wid * BATCHES_PER_TILE

        pltpu.sync_copy(idx_hbm.at[wid], idx_v.at[pl.ds(0, TOKENS_PER_TILE)])

        def issue_gathers(b, j):
            base = b * SEQ
            vecs = [idx_v[pl.ds(base + off, 16)] for off in (0, 16, 32, 48)]
            for t in range(SEQ):
                v, lane = divmod(t, 16)
                pltpu.async_copy(
                    table_hbm.at[pl.ds(vecs[v][lane], 1)],
                    bufs.at[j, pl.ds(t, 1)], gsems[j])

        for j in range(NBUF):
            issue_gathers(j, j)

        def body(i, carry):
            del carry
            for j in range(NBUF):
                b = NBUF * i + j
                pltpu.make_async_copy(out_hbm.at[0], bufs.at[0], gsems[j]).wait()
                pltpu.async_copy(bufs.at[j], out_hbm.at[b_first + b], wsems[j])
            for j in range(NBUF):
                @pl.when(i < BATCHES_PER_TILE // NBUF - 1)
                def _():
                    pltpu.make_async_copy(
                        bufs.at[0], out_hbm.at[0], wsems[j]).wait()
                    issue_gathers(NBUF * (i + 1) + j, j)
            return 0

        lax.fori_loop(0, BATCHES_PER_TILE // NBUF, body, 0, unroll=False)
        for j in range(NBUF):
            pltpu.make_async_copy(bufs.at[0], out_hbm.at[0], wsems[j]).wait()

    return k


def kernel(indices, table):
    idx = indices.astype(jnp.int32).reshape(NW, TOKENS_PER_TILE)
    return _build_gather_kernel()(idx, table)
